# baseline (speedup 1.0000x reference)
import functools

import jax
import jax.numpy as jnp
from jax import lax
from jax.experimental import pallas as pl
from jax.experimental.pallas import tpu as pltpu
from jax.experimental.pallas import tpu_sc as plsc

_EMBED_DIM = 32
_PACK = 4
_LINE = _EMBED_DIM * _PACK
_NUM_CORES = 2
_NUM_SUBCORES = 16
_NUM_WORKERS = _NUM_CORES * _NUM_SUBCORES
_LANES = 16


def _make_gather(batch: int, hist: int, nlines: int):
    assert batch % _NUM_WORKERS == 0
    blk = batch // _NUM_WORKERS
    assert blk % _LANES == 0
    ngroups = blk // _LANES
    mesh = plsc.VectorSubcoreMesh(core_axis_name="c", subcore_axis_name="s")

    @functools.partial(
        pl.kernel,
        mesh=mesh,
        compiler_params=pltpu.CompilerParams(
            use_tc_tiling_on_sc=True, needs_layout_passes=False),
        out_type=jax.ShapeDtypeStruct((hist, _EMBED_DIM, batch), jnp.float32),
        scratch_types=[
            pltpu.VMEM((hist, blk), jnp.int32),
            pltpu.VMEM((2, blk), jnp.int32),
            pltpu.VMEM((2, blk, _LINE), jnp.float32),
            pltpu.VMEM((2, _EMBED_DIM, blk), jnp.float32),
            pltpu.SemaphoreType.DMA((2,)),
            pltpu.SemaphoreType.DMA((2,)),
        ],
    )
    def gather_kernel(table_hbm, idx_hbm, out_hbm,
                      idx_v, line_v, g_v, t_v, g_sem, w_sem):
        wid = lax.axis_index("s") * _NUM_CORES + lax.axis_index("c")
        col0 = wid * blk
        pltpu.sync_copy(idx_hbm.at[:, pl.ds(col0, blk)], idx_v)

        def compute_lines(h, b):
            for g in range(ngroups):
                sl = pl.ds(g * _LANES, _LANES)
                line_v[b, sl] = lax.shift_right_logical(idx_v[h, sl], 2)

        def gather(b):
            return pltpu.make_async_copy(
                table_hbm.at[line_v.at[b]], g_v.at[b], g_sem.at[b])

        def writeback(h, b):
            return pltpu.make_async_copy(
                t_v.at[b], out_hbm.at[h, :, pl.ds(col0, blk)], w_sem.at[b])

        compute_lines(0, 0)
        gather(0).start()
        compute_lines(1, 1)
        gather(1).start()

        def body(h, carry):
            b = h % 2
            @pl.when(h >= 2)
            def _():
                writeback(h - 2, b).wait()
            gather(b).wait()
            gb = g_v.at[b]
            tb = t_v.at[b]
            for g in range(ngroups):
                sl = pl.ds(g * _LANES, _LANES)
                rows = lax.iota(jnp.int32, _LANES) + g * _LANES
                q = lax.shift_left(idx_v[h, sl] & 3, 5)
                for d in range(_EMBED_DIM):
                    vec = plsc.load_gather(gb, [rows, q + d])
                    tb[d, sl] = vec
            writeback(h, b).start()
            @pl.when(h + 2 < hist)
            def _():
                compute_lines(h + 2, b)
                gather(b).start()
            return carry

        lax.fori_loop(0, hist, body, 0)
        writeback(hist - 2, hist % 2).wait()
        writeback(hist - 1, (hist + 1) % 2).wait()

    return gather_kernel


def kernel(words, chars, table):
    del chars
    batch, hist = words.shape
    vocab = table.shape[0]
    packed = table.reshape(vocab // _PACK, _LINE)
    out = _make_gather(batch, hist, vocab // _PACK)(packed, words.T)
    return out.transpose(2, 0, 1)


# speedup vs baseline: 1.1446x; 1.1446x over previous
import functools

import jax
import jax.numpy as jnp
from jax import lax
from jax.experimental import pallas as pl
from jax.experimental.pallas import tpu as pltpu
from jax.experimental.pallas import tpu_sc as plsc

_EMBED_DIM = 32
_PAD_DIM = 128
_NUM_CORES = 2
_NUM_SUBCORES = 16
_NUM_WORKERS = _NUM_CORES * _NUM_SUBCORES
_LANES = 16


def _make_gather(batch: int, hist: int):
    assert batch % _NUM_WORKERS == 0
    blk = batch // _NUM_WORKERS
    assert blk % _LANES == 0
    mesh = plsc.VectorSubcoreMesh(core_axis_name="c", subcore_axis_name="s")

    @functools.partial(
        pl.kernel,
        mesh=mesh,
        compiler_params=pltpu.CompilerParams(
            use_tc_tiling_on_sc=True, needs_layout_passes=False),
        out_type=jax.ShapeDtypeStruct((hist, _EMBED_DIM, batch), jnp.float32),
        scratch_types=[
            pltpu.VMEM((hist, blk), jnp.int32),
            pltpu.VMEM((2, blk, _PAD_DIM), jnp.float32),
            pltpu.VMEM((2, _EMBED_DIM, blk), jnp.float32),
            pltpu.SemaphoreType.DMA((2,)),
            pltpu.SemaphoreType.DMA((2,)),
        ],
    )
    def gather_kernel(table_hbm, idx_hbm, out_hbm, idx_v, g_v, t_v, g_sem, w_sem):
        wid = lax.axis_index("s") * _NUM_CORES + lax.axis_index("c")
        col0 = wid * blk
        pltpu.sync_copy(idx_hbm.at[:, pl.ds(col0, blk)], idx_v)

        def gather(h, b):
            return pltpu.make_async_copy(
                table_hbm.at[idx_v.at[h]], g_v.at[b], g_sem.at[b])

        def writeback(h, b):
            return pltpu.make_async_copy(
                t_v.at[b], out_hbm.at[h, :, pl.ds(col0, blk)], w_sem.at[b])

        gather(0, 0).start()
        gather(1, 1).start()

        def body(h, carry):
            b = h % 2
            @pl.when(h >= 2)
            def _():
                writeback(h - 2, b).wait()
            gather(h, b).wait()
            gb = g_v.at[b]
            tb = t_v.at[b]

            @plsc.parallel_loop(0, blk, 1, unroll=8)
            def _(r):
                cb = jnp.full((_LANES,), 0, jnp.int32) + r
                for k in range(_EMBED_DIM // _LANES):
                    vec = gb[r, pl.ds(k * _LANES, _LANES)]
                    plsc.store_scatter(
                        tb, [lax.iota(jnp.int32, _LANES) + k * _LANES, cb], vec)

            writeback(h, b).start()
            @pl.when(h + 2 < hist)
            def _():
                gather(h + 2, b).start()
            return carry

        lax.fori_loop(0, hist, body, 0)
        writeback(hist - 2, hist % 2).wait()
        writeback(hist - 1, (hist + 1) % 2).wait()

    return gather_kernel


def kernel(words, chars, table):
    del chars
    batch, hist = words.shape
    tp = jnp.pad(table, ((0, 0), (0, _PAD_DIM - _EMBED_DIM)))
    out = _make_gather(batch, hist)(tp, words.T)
    return out.transpose(2, 0, 1)
